# Initial kernel scaffold; baseline (speedup 1.0000x reference)
import jax
import jax.numpy as jnp
from jax.experimental import pallas as pl


def kernel(x, embed_weight):
    raise NotImplementedError("write your pallas kernel here")


# speedup vs baseline: 2.7033x; 2.7033x over previous
import functools

import jax
import jax.numpy as jnp
from jax.experimental import pallas as pl
from jax.experimental.pallas import tpu as pltpu

N_ROWS = 512
N_CODES = 8192
DIM = 32
CHUNK = 1024
N_CHUNKS = N_CODES // CHUNK


def _vq_kernel(xf_ref, et_ref, e_ref, out_ref, best_ref, bidx_ref):
    s = pl.program_id(0)
    is_dist = s < N_CHUNKS
    c = jnp.where(is_dist, s, s - N_CHUNKS)

    @pl.when(is_dist)
    def _distance_step():
        xm = xf_ref[:, :]
        ec = et_ref[:, :]
        d = None
        for r in range(4):
            t = []
            for i in range(8):
                k = 8 * r + i
                dd = xm[:, k:k + 1] - ec[k:k + 1, :]
                t.append(dd * dd)
            b0 = t[0] + t[4]
            b1 = t[1] + t[5]
            b2 = t[2] + t[6]
            b3 = t[3] + t[7]
            c0 = b0 + b2
            c1 = b1 + b3
            sgrp = c0 + c1
            d = sgrp if d is None else d + sgrp
        m = jnp.min(d, axis=1, keepdims=True)
        iota = jax.lax.broadcasted_iota(jnp.int32, (N_ROWS, CHUNK), 1)
        masked = jnp.where(d == m, iota, N_CODES)
        a = jnp.min(masked, axis=1, keepdims=True) + s * CHUNK

        @pl.when(s == 0)
        def _init():
            best_ref[:, :] = m
            bidx_ref[:, :] = a

        @pl.when(s > 0)
        def _update():
            prev_m = best_ref[:, :]
            prev_a = bidx_ref[:, :]
            upd = m < prev_m
            best_ref[:, :] = jnp.where(upd, m, prev_m)
            bidx_ref[:, :] = jnp.where(upd, a, prev_a)

    @pl.when(jnp.logical_not(is_dist))
    def _gather_step():
        bi = bidx_ref[:, :]
        iota = jax.lax.broadcasted_iota(jnp.int32, (N_ROWS, CHUNK), 1)
        oh = (bi == (iota + c * CHUNK)).astype(jnp.float32)
        part = jnp.dot(oh, e_ref[:, :], preferred_element_type=jnp.float32)

        @pl.when(s == N_CHUNKS)
        def _first():
            out_ref[:, :] = part

        @pl.when(s > N_CHUNKS)
        def _acc():
            out_ref[:, :] = out_ref[:, :] + part


@jax.jit
def kernel(x, embed_weight):
    ori_shape = x.shape
    b, ch, h, w = ori_shape
    xf = jnp.transpose(x, (0, 2, 3, 1)).reshape(b * h * w, ch)
    et = embed_weight.T

    q = pl.pallas_call(
        _vq_kernel,
        grid=(2 * N_CHUNKS,),
        in_specs=[
            pl.BlockSpec((N_ROWS, DIM), lambda s: (0, 0)),
            pl.BlockSpec((DIM, CHUNK), lambda s: (0, s % N_CHUNKS)),
            pl.BlockSpec((CHUNK, DIM), lambda s: (s % N_CHUNKS, 0)),
        ],
        out_specs=pl.BlockSpec((N_ROWS, DIM), lambda s: (0, 0)),
        out_shape=jax.ShapeDtypeStruct((N_ROWS, DIM), jnp.float32),
        scratch_shapes=[
            pltpu.VMEM((N_ROWS, 1), jnp.float32),
            pltpu.VMEM((N_ROWS, 1), jnp.int32),
        ],
    )(xf, et, embed_weight)

    return q.reshape(ori_shape)
